# baseline (speedup 1.0000x reference)
import functools

import jax
import jax.numpy as jnp
from jax import lax
from jax.experimental import pallas as pl
from jax.experimental.pallas import tpu as pltpu, tpu_sc as plsc

NUM_ACTIONS = 100000
D = 128
B = 16384
HALF = B // 2
NC, NS = 2, 16
NW = NC * NS
ROWS_PER_W = HALF // NW
CHUNK = 128
NCHUNK = ROWS_PER_W // CHUNK
ZROWS = 64

_mesh = plsc.VectorSubcoreMesh(core_axis_name="c", subcore_axis_name="s")


@functools.partial(
    pl.kernel,
    out_type=jax.ShapeDtypeStruct((B, D), jnp.float32),
    mesh=_mesh,
    scratch_types=[
        pltpu.VMEM((CHUNK,), jnp.int32),
        pltpu.VMEM((CHUNK,), jnp.int32),
        pltpu.VMEM((CHUNK, D), jnp.float32),
        pltpu.VMEM((CHUNK, D), jnp.float32),
        pltpu.VMEM((ZROWS, D), jnp.float32),
        pltpu.SemaphoreType.DMA,
        pltpu.SemaphoreType.DMA,
        pltpu.SemaphoreType.DMA,
    ],
)
def _embed_gather(idx_hbm, table_hbm, out_hbm,
                  idx0, idx1, rows0, rows1, zbuf, sem0, sem1, semz):
    wid = lax.axis_index("s") * NC + lax.axis_index("c")
    base = wid * ROWS_PER_W

    idxb = (idx0, idx1)
    rowsb = (rows0, rows1)
    sems = (sem0, sem1)

    iload = [
        pltpu.async_copy(
            idx_hbm.at[pl.ds(HALF + base + j * CHUNK, CHUNK)], idxb[j], sems[j])
        for j in range(NCHUNK)
    ]

    z16 = jnp.zeros((16,), jnp.float32)

    def _zfill(i, carry):
        for k in range(D // 16):
            zbuf[i, pl.ds(k * 16, 16)] = z16
        return carry

    lax.fori_loop(0, ZROWS, _zfill, 0)

    gathers = []
    for j in range(NCHUNK):
        iload[j].wait()
        gathers.append(
            pltpu.async_copy(table_hbm.at[idxb[j]], rowsb[j], sems[j]))

    zwrites = [
        pltpu.async_copy(zbuf, out_hbm.at[pl.ds(base + z * ZROWS, ZROWS)], semz)
        for z in range(ROWS_PER_W // ZROWS)
    ]

    rwrites = []
    for j in range(NCHUNK):
        gathers[j].wait()
        rwrites.append(pltpu.async_copy(
            rowsb[j], out_hbm.at[pl.ds(HALF + base + j * CHUNK, CHUNK)],
            sems[j]))
    for c in zwrites:
        c.wait()
    for c in rwrites:
        c.wait()


def kernel(input, action_embedding):
    idx_all = input.reshape(B).astype(jnp.int32)
    out = _embed_gather(idx_all, action_embedding)
    return out[None]
